# baseline (speedup 1.0000x reference)
import functools

import jax
import jax.numpy as jnp
from jax import lax
from jax.experimental import pallas as pl
from jax.experimental.pallas import tpu as pltpu
from jax.experimental.pallas import tpu_sc as plsc

VOCAB = 1000000
EMBED_DIM = 32
BATCH = 16384
HIST = 50

_info = plsc.get_sparse_core_info()
NUM_CORES = _info.num_cores
NUM_SUBCORES = _info.num_subcores
NW = NUM_CORES * NUM_SUBCORES

VPAD = 1000064
PACK = 4
RM_ROWS = VPAD // PACK
W1 = 128
N1_FULL = VOCAB // W1
TAIL0 = N1_FULL * W1
TAILW = VOCAB - TAIL0
W2 = 512
LANES = 16


def _relayout_body(tab_hbm, tail_hbm, table_rm, tin, rout, tailv):
    c = lax.axis_index("c")
    s = lax.axis_index("s")
    wid = s * NUM_CORES + c
    iota = lax.iota(jnp.int32, LANES)
    row_pat = lax.shift_right_logical(iota, 2)
    col_pat = (iota & 3) * EMBED_DIM

    def slab_loop(k, carry):
        c0 = (wid + k * NW) * W1
        pltpu.sync_copy(tab_hbm.at[:, pl.ds(c0, W1)], tin)

        def e_loop(e, carry2):
            for l0 in range(0, W1, LANES):
                vec = tin[e, pl.ds(l0, LANES)]
                plsc.store_scatter(
                    rout, [row_pat + (l0 // PACK), col_pat + e], vec)
            return carry2

        lax.fori_loop(0, EMBED_DIM, e_loop, 0)
        r0 = pl.multiple_of(c0 // PACK, 32)
        pltpu.sync_copy(rout, table_rm.at[pl.ds(r0, W1 // PACK)])
        return carry

    nk = (N1_FULL - wid + NW - 1) // NW
    lax.fori_loop(0, nk, slab_loop, 0)

    @pl.when(wid == 0)
    def _tail():
        pltpu.sync_copy(tail_hbm, tailv)

        def v_loop(v, carry):
            r = lax.shift_right_logical(v, 2)
            c0 = (v & 3) * EMBED_DIM
            for q in range(0, EMBED_DIM, LANES):
                vec = tailv[v, pl.ds(q, LANES)]
                plsc.store_scatter(
                    rout,
                    [jnp.full((LANES,), r, jnp.int32),
                     iota + (c0 + q)],
                    vec)
            return carry

        lax.fori_loop(0, TAILW, v_loop, 0)
        pltpu.sync_copy(rout.at[pl.ds(0, TAILW // PACK)],
                        table_rm.at[pl.ds(TAIL0 // PACK, TAILW // PACK)])


def _gather_body(idx_hbm, table_rm, out_hbm, idx_all, idx4, g4, t, gsem):
    c = lax.axis_index("c")
    s = lax.axis_index("s")
    wid = s * NUM_CORES + c
    iota = lax.iota(jnp.int32, LANES)
    b0 = wid * W2
    pltpu.sync_copy(idx_hbm.at[:, pl.ds(b0, W2)], idx_all)

    def h_loop(h, carry):
        def j_loop(j, carry2):
            vec = idx_all[h, pl.ds(j * LANES, LANES)]
            idx4[pl.ds(j * LANES, LANES)] = lax.shift_right_logical(vec, 2)
            return carry2

        lax.fori_loop(0, W2 // LANES, j_loop, 0)
        pltpu.async_copy(table_rm.at[idx4], g4, gsem).wait()

        def p_loop(p, carry2):
            rows = iota + p * LANES
            colbase = (idx_all[h, pl.ds(p * LANES, LANES)] & 3) * EMBED_DIM
            for e in range(EMBED_DIM):
                vec = plsc.load_gather(g4, [rows, colbase + e])
                t[e, pl.ds(p * LANES, LANES)] = vec
            return carry2

        lax.fori_loop(0, W2 // LANES, p_loop, 0)
        pltpu.sync_copy(t, out_hbm.at[h, :, pl.ds(b0, W2)])
        return carry

    lax.fori_loop(0, HIST, h_loop, 0)


@jax.jit
def _embed_lookup(idx_t, tab_t, tail_pad):
    mesh = plsc.VectorSubcoreMesh(core_axis_name="c", subcore_axis_name="s")
    relayout = functools.partial(
        pl.kernel,
        mesh=mesh,
        out_type=jax.ShapeDtypeStruct((RM_ROWS, 128), jnp.float32),
        scratch_types=[
            pltpu.VMEM((EMBED_DIM, W1), jnp.float32),
            pltpu.VMEM((W1 // PACK, 128), jnp.float32),
            pltpu.VMEM((TAILW, 128), jnp.float32),
        ],
        compiler_params=pltpu.CompilerParams(needs_layout_passes=False),
    )(_relayout_body)
    table_rm = relayout(tab_t, tail_pad)

    gather = functools.partial(
        pl.kernel,
        mesh=mesh,
        out_type=jax.ShapeDtypeStruct((HIST, EMBED_DIM, BATCH), jnp.float32),
        scratch_types=[
            pltpu.VMEM((HIST, W2), jnp.int32),
            pltpu.VMEM((W2,), jnp.int32),
            pltpu.VMEM((W2, 128), jnp.float32),
            pltpu.VMEM((EMBED_DIM, W2), jnp.float32),
            pltpu.SemaphoreType.DMA,
        ],
        compiler_params=pltpu.CompilerParams(needs_layout_passes=False),
    )(_gather_body)
    return gather(idx_t, table_rm)


def kernel(indices, table):
    tail_pad = jnp.pad(table[TAIL0:], ((0, 0), (0, 128 - EMBED_DIM)))
    out_t = _embed_lookup(indices.T, table.T, tail_pad)
    return out_t.transpose(2, 0, 1)


# speedup vs baseline: 1.1051x; 1.1051x over previous
import functools

import jax
import jax.numpy as jnp
from jax import lax
from jax.experimental import pallas as pl
from jax.experimental.pallas import tpu as pltpu
from jax.experimental.pallas import tpu_sc as plsc

VOCAB = 1000000
EMBED_DIM = 32
BATCH = 16384
HIST = 50

_info = plsc.get_sparse_core_info()
NUM_CORES = _info.num_cores
NUM_SUBCORES = _info.num_subcores
NW = NUM_CORES * NUM_SUBCORES

VPAD = 1000064
PACK = 4
RM_ROWS = VPAD // PACK
W1 = 128
N1_FULL = VOCAB // W1
TAIL0 = N1_FULL * W1
TAILW = VOCAB - TAIL0
NK = 246
W2 = 256
NUNIT = HIST * 2
LANES = 16


def _relayout_body(tab_hbm, tail_hbm, table_rm, tin, rout, tailv, isem, osem):
    wid = lax.axis_index("s") * NUM_CORES + lax.axis_index("c")
    _IOTA = lax.iota(jnp.int32, LANES)

    def slab(k):
        return jnp.minimum(wid + k * NW, N1_FULL - PACK + (wid & 3))

    def in_desc(k, b):
        c0 = slab(k) * W1
        return pltpu.make_async_copy(
            tab_hbm.at[:, pl.ds(c0, W1)], tin.at[b], isem.at[b])

    def out_desc(k, b):
        r0 = pl.multiple_of(slab(k) * (W1 // PACK), 32)
        return pltpu.make_async_copy(
            rout.at[b], table_rm.at[pl.ds(r0, W1 // PACK)], osem.at[b])

    def transpose(b):
        for r in range(W1 // PACK):
            for ci in range(8):
                rows = _IOTA + (ci % 2) * LANES
                cols = jnp.full((LANES,), PACK * r + ci // 2, jnp.int32)
                vec = plsc.load_gather(tin.at[b], [rows, cols])
                rout[b, r, pl.ds(ci * LANES, LANES)] = vec

    in_desc(0, 0).start()
    in_desc(1, 1).start()

    def pair(kk, carry):
        for b in range(2):
            k = 2 * kk + b
            in_desc(k, b).wait()

            @pl.when(kk > 0)
            def _wait_prev():
                out_desc(k - 2, b).wait()

            transpose(b)
            out_desc(k, b).start()

            @pl.when(kk < NK // 2 - 1)
            def _next_in():
                in_desc(k + 2, b).start()

        return carry

    lax.fori_loop(0, NK // 2, pair, 0)
    out_desc(NK - 2, 0).wait()
    out_desc(NK - 1, 1).wait()

    @pl.when(wid == 0)
    def _tail():
        pltpu.sync_copy(tail_hbm, tailv)

        def v_loop(v, carry):
            r = lax.shift_right_logical(v, 2)
            cb = (v & 3) * EMBED_DIM
            for q in range(0, EMBED_DIM, LANES):
                vec = tailv[v, pl.ds(q, LANES)]
                plsc.store_scatter(
                    rout.at[0],
                    [jnp.full((LANES,), r, jnp.int32), _IOTA + (cb + q)],
                    vec)
            return carry

        lax.fori_loop(0, TAILW, v_loop, 0)
        pltpu.sync_copy(rout.at[0, pl.ds(0, TAILW // PACK)],
                        table_rm.at[pl.ds(TAIL0 // PACK, TAILW // PACK)])


def _gather_body(idx_hbm, table_rm, out_hbm, idx_all, idx4a, idx4b, g4, t, gsem):
    wid = lax.axis_index("s") * NUM_CORES + lax.axis_index("c")
    _IOTA = lax.iota(jnp.int32, LANES)
    b0 = wid * (2 * W2)
    pltpu.sync_copy(idx_hbm.at[:, pl.ds(b0, 2 * W2)], idx_all)

    def prep(u, b):
        h = lax.shift_right_logical(u, 1)
        boff = (u & 1) * W2
        idx4 = idx4a if b == 0 else idx4b
        for j in range(W2 // LANES):
            vec = idx_all[h, pl.ds(boff + j * LANES, LANES)]
            idx4[pl.ds(j * LANES, LANES)] = lax.shift_right_logical(vec, 2)

    def g_desc(b):
        idx4 = idx4a if b == 0 else idx4b
        return pltpu.make_async_copy(
            table_rm.at[idx4], g4.at[b], gsem.at[b])

    def unit(u, b):
        h = lax.shift_right_logical(u, 1)
        boff = (u & 1) * W2

        def p_loop(p, carry):
            rows = _IOTA + p * LANES
            colb = (idx_all[h, pl.ds(boff + p * LANES, LANES)] & 3) * EMBED_DIM
            for e in range(EMBED_DIM):
                vec = plsc.load_gather(g4.at[b], [rows, colb + e])
                t[e, pl.ds(p * LANES, LANES)] = vec
            return carry

        lax.fori_loop(0, W2 // LANES, p_loop, 0)
        pltpu.sync_copy(t, out_hbm.at[h, :, pl.ds(b0 + boff, W2)])

    prep(0, 0)
    g_desc(0).start()
    prep(1, 1)
    g_desc(1).start()

    def upair(uu, carry):
        for b in range(2):
            u = 2 * uu + b
            g_desc(b).wait()
            unit(u, b)

            @pl.when(uu < NUNIT // 2 - 1)
            def _next():
                prep(u + 2, b)
                g_desc(b).start()

        return carry

    lax.fori_loop(0, NUNIT // 2, upair, 0)


@jax.jit
def _embed_lookup(idx_t, tab_t, tail_pad):
    mesh = plsc.VectorSubcoreMesh(core_axis_name="c", subcore_axis_name="s")
    relayout = functools.partial(
        pl.kernel,
        mesh=mesh,
        out_type=jax.ShapeDtypeStruct((RM_ROWS, 128), jnp.float32),
        scratch_types=[
            pltpu.VMEM((2, EMBED_DIM, W1), jnp.float32),
            pltpu.VMEM((2, W1 // PACK, 128), jnp.float32),
            pltpu.VMEM((TAILW, 128), jnp.float32),
            pltpu.SemaphoreType.DMA((2,)),
            pltpu.SemaphoreType.DMA((2,)),
        ],
        compiler_params=pltpu.CompilerParams(needs_layout_passes=False),
    )(_relayout_body)
    table_rm = relayout(tab_t, tail_pad)

    gather = functools.partial(
        pl.kernel,
        mesh=mesh,
        out_type=jax.ShapeDtypeStruct((HIST, EMBED_DIM, BATCH), jnp.float32),
        scratch_types=[
            pltpu.VMEM((HIST, 2 * W2), jnp.int32),
            pltpu.VMEM((W2,), jnp.int32),
            pltpu.VMEM((W2,), jnp.int32),
            pltpu.VMEM((2, W2, 128), jnp.float32),
            pltpu.VMEM((EMBED_DIM, W2), jnp.float32),
            pltpu.SemaphoreType.DMA((2,)),
        ],
        compiler_params=pltpu.CompilerParams(needs_layout_passes=False),
    )(_gather_body)
    return gather(idx_t, table_rm)


def kernel(indices, table):
    tail_pad = jnp.pad(table[TAIL0:], ((0, 0), (0, 128 - EMBED_DIM)))
    out_t = _embed_lookup(indices.T, table.T, tail_pad)
    return out_t.transpose(2, 0, 1)


# speedup vs baseline: 2.2820x; 2.0650x over previous
import functools

import jax
import jax.numpy as jnp
from jax import lax
from jax.experimental import pallas as pl
from jax.experimental.pallas import tpu as pltpu
from jax.experimental.pallas import tpu_sc as plsc

VOCAB = 1000000
EMBED_DIM = 32
BATCH = 16384
HIST = 50

_info = plsc.get_sparse_core_info()
NUM_CORES = _info.num_cores
NUM_SUBCORES = _info.num_subcores
NW = NUM_CORES * NUM_SUBCORES

VPAD = 1000064
PACK = 4
RM_ROWS = VPAD // PACK
W1 = 128
N1_FULL = VOCAB // W1
TAIL0 = N1_FULL * W1
TAILW = VOCAB - TAIL0
NK = 246
W2 = 256
NUNIT = HIST * 2
LANES = 16


def _relayout_body(tab_hbm, tail_hbm, table_rm, tin, rout, tailv, isem, osem):
    wid = lax.axis_index("s") * NUM_CORES + lax.axis_index("c")
    _IOTA = lax.iota(jnp.int32, LANES)

    def slab(k):
        return jnp.minimum(wid + k * NW, N1_FULL - PACK + (wid & 3))

    def in_desc(k, b):
        c0 = slab(k) * W1
        return pltpu.make_async_copy(
            tab_hbm.at[:, pl.ds(c0, W1)], tin.at[b], isem.at[b])

    def out_desc(k, b):
        r0 = pl.multiple_of(slab(k) * (W1 // PACK), 32)
        return pltpu.make_async_copy(
            rout.at[b], table_rm.at[pl.ds(r0, W1 // PACK)], osem.at[b])

    PERM = [(_IOTA + d) % LANES for d in range(LANES)]
    RQ = lax.shift_right_logical(_IOTA, 2)
    CQ = (_IOTA & 3) * EMBED_DIM

    def transpose(b):
        for r0 in range(0, W1 // PACK, PACK):
            cols_src = _IOTA + PACK * r0
            rows_dst = RQ + r0
            for e0 in range(0, EMBED_DIM, LANES):
                for d in range(LANES):
                    epat = PERM[d] + e0
                    vec = plsc.load_gather(tin.at[b], [epat, cols_src])
                    plsc.store_scatter(rout.at[b], [rows_dst, epat + CQ], vec)

    in_desc(0, 0).start()
    in_desc(1, 1).start()

    def pair(kk, carry):
        for b in range(2):
            k = 2 * kk + b
            in_desc(k, b).wait()

            @pl.when(kk > 0)
            def _wait_prev():
                out_desc(k - 2, b).wait()

            transpose(b)
            out_desc(k, b).start()

            @pl.when(kk < NK // 2 - 1)
            def _next_in():
                in_desc(k + 2, b).start()

        return carry

    lax.fori_loop(0, NK // 2, pair, 0)
    out_desc(NK - 2, 0).wait()
    out_desc(NK - 1, 1).wait()

    @pl.when(wid == 0)
    def _tail():
        pltpu.sync_copy(tail_hbm, tailv)

        def v_loop(v, carry):
            r = lax.shift_right_logical(v, 2)
            cb = (v & 3) * EMBED_DIM
            for q in range(0, EMBED_DIM, LANES):
                vec = tailv[v, pl.ds(q, LANES)]
                plsc.store_scatter(
                    rout.at[0],
                    [jnp.full((LANES,), r, jnp.int32), _IOTA + (cb + q)],
                    vec)
            return carry

        lax.fori_loop(0, TAILW, v_loop, 0)
        pltpu.sync_copy(rout.at[0, pl.ds(0, TAILW // PACK)],
                        table_rm.at[pl.ds(TAIL0 // PACK, TAILW // PACK)])


def _gather_body(idx_hbm, table_rm, out_hbm, idx_all, idx4a, idx4b, g4, t, gsem):
    wid = lax.axis_index("s") * NUM_CORES + lax.axis_index("c")
    _IOTA = lax.iota(jnp.int32, LANES)
    PERM = [(_IOTA + d) % LANES for d in range(LANES)]
    b0 = wid * (2 * W2)
    pltpu.sync_copy(idx_hbm.at[:, pl.ds(b0, 2 * W2)], idx_all)

    def prep(u, b):
        h = lax.shift_right_logical(u, 1)
        boff = (u & 1) * W2
        idx4 = idx4a if b == 0 else idx4b
        for j in range(W2 // LANES):
            vec = idx_all[h, pl.ds(boff + j * LANES, LANES)]
            idx4[pl.ds(j * LANES, LANES)] = lax.shift_right_logical(vec, 2)

    def g_desc(b):
        idx4 = idx4a if b == 0 else idx4b
        return pltpu.make_async_copy(
            table_rm.at[idx4], g4.at[b], gsem.at[b])

    def unit(u, b):
        h = lax.shift_right_logical(u, 1)
        boff = (u & 1) * W2

        def p_loop(p, carry):
            rows = _IOTA + p * LANES
            colb = (idx_all[h, pl.ds(boff + p * LANES, LANES)] & 3) * EMBED_DIM
            for e0 in range(0, EMBED_DIM, LANES):
                for d in range(LANES):
                    epat = PERM[d] + e0
                    vec = plsc.load_gather(g4.at[b], [rows, colb + epat])
                    plsc.store_scatter(t, [epat, rows], vec)
            return carry

        lax.fori_loop(0, W2 // LANES, p_loop, 0)
        pltpu.sync_copy(t, out_hbm.at[h, :, pl.ds(b0 + boff, W2)])

    prep(0, 0)
    g_desc(0).start()
    prep(1, 1)
    g_desc(1).start()

    def upair(uu, carry):
        for b in range(2):
            u = 2 * uu + b
            g_desc(b).wait()
            unit(u, b)

            @pl.when(uu < NUNIT // 2 - 1)
            def _next():
                prep(u + 2, b)
                g_desc(b).start()

        return carry

    lax.fori_loop(0, NUNIT // 2, upair, 0)


@jax.jit
def _embed_lookup(idx_t, tab_t, tail_pad):
    mesh = plsc.VectorSubcoreMesh(core_axis_name="c", subcore_axis_name="s")
    relayout = functools.partial(
        pl.kernel,
        mesh=mesh,
        out_type=jax.ShapeDtypeStruct((RM_ROWS, 128), jnp.float32),
        scratch_types=[
            pltpu.VMEM((2, EMBED_DIM, W1), jnp.float32),
            pltpu.VMEM((2, W1 // PACK, 128), jnp.float32),
            pltpu.VMEM((TAILW, 128), jnp.float32),
            pltpu.SemaphoreType.DMA((2,)),
            pltpu.SemaphoreType.DMA((2,)),
        ],
        compiler_params=pltpu.CompilerParams(needs_layout_passes=False),
    )(_relayout_body)
    table_rm = relayout(tab_t, tail_pad)

    gather = functools.partial(
        pl.kernel,
        mesh=mesh,
        out_type=jax.ShapeDtypeStruct((HIST, EMBED_DIM, BATCH), jnp.float32),
        scratch_types=[
            pltpu.VMEM((HIST, 2 * W2), jnp.int32),
            pltpu.VMEM((W2,), jnp.int32),
            pltpu.VMEM((W2,), jnp.int32),
            pltpu.VMEM((2, W2, 128), jnp.float32),
            pltpu.VMEM((EMBED_DIM, W2), jnp.float32),
            pltpu.SemaphoreType.DMA((2,)),
        ],
        compiler_params=pltpu.CompilerParams(needs_layout_passes=False),
    )(_gather_body)
    return gather(idx_t, table_rm)


def kernel(indices, table):
    tail_pad = jnp.pad(table[TAIL0:], ((0, 0), (0, 128 - EMBED_DIM)))
    out_t = _embed_lookup(indices.T, table.T, tail_pad)
    return out_t.transpose(2, 0, 1)


# speedup vs baseline: 3.0767x; 1.3483x over previous
import functools

import jax
import jax.numpy as jnp
from jax import lax
from jax.experimental import pallas as pl
from jax.experimental.pallas import tpu as pltpu
from jax.experimental.pallas import tpu_sc as plsc

VOCAB = 1000000
EMBED_DIM = 32
BATCH = 16384
HIST = 50

_info = plsc.get_sparse_core_info()
NUM_CORES = _info.num_cores
NUM_SUBCORES = _info.num_subcores
NW = NUM_CORES * NUM_SUBCORES

VPAD = 1000064
PACK = 4
RM_ROWS = VPAD // PACK
W1 = 256
N1_FULL = VOCAB // W1
TAIL0 = N1_FULL * W1
TAILW = VOCAB - TAIL0
NK = 124
W2 = 256
NUNIT = HIST * 2
LANES = 16


def _relayout_body(tab_hbm, tail_hbm, table_rm, tin, rout, tailv, permv, isem, osem):
    wid = lax.axis_index("s") * NUM_CORES + lax.axis_index("c")
    _IOTA = lax.iota(jnp.int32, LANES)

    def slab(k):
        return jnp.minimum(wid + k * NW, N1_FULL - 2 + (wid & 1))

    def in_desc(k, b):
        c0 = slab(k) * W1
        return pltpu.make_async_copy(
            tab_hbm.at[:, pl.ds(c0, W1)], tin.at[b], isem.at[b])

    def out_desc(k, b):
        r0 = pl.multiple_of(slab(k) * (W1 // PACK), 64)
        return pltpu.make_async_copy(
            rout.at[b], table_rm.at[pl.ds(r0, W1 // PACK)], osem.at[b])

    PERM = [(_IOTA + d) % LANES for d in range(LANES)]
    RQ = lax.shift_right_logical(_IOTA, 2)
    CQ = (_IOTA & 3) * EMBED_DIM

    for d in range(LANES):
        permv[d, pl.ds(0, LANES)] = PERM[d]

    def transpose(b):
        def d_loop(d, carry):
            pr = permv[d, pl.ds(0, LANES)]
            for e0 in range(0, EMBED_DIM, LANES):
                epat = pr + e0
                epatc = epat + CQ
                for r0 in range(0, W1 // PACK, PACK):
                    vec = plsc.load_gather(tin.at[b], [epat, _IOTA + PACK * r0])
                    plsc.store_scatter(rout.at[b], [RQ + r0, epatc], vec)
            return carry

        lax.fori_loop(0, LANES, d_loop, 0)

    in_desc(0, 0).start()
    in_desc(1, 1).start()

    def pair(kk, carry):
        for b in range(2):
            k = 2 * kk + b
            in_desc(k, b).wait()

            @pl.when(kk > 0)
            def _wait_prev():
                out_desc(k - 2, b).wait()

            transpose(b)
            out_desc(k, b).start()

            @pl.when(kk < NK // 2 - 1)
            def _next_in():
                in_desc(k + 2, b).start()

        return carry

    lax.fori_loop(0, NK // 2, pair, 0)
    out_desc(NK - 2, 0).wait()
    out_desc(NK - 1, 1).wait()

    @pl.when(wid == 0)
    def _tail():
        pltpu.sync_copy(tail_hbm, tailv)

        def v_loop(v, carry):
            r = lax.shift_right_logical(v, 2)
            cb = (v & 3) * EMBED_DIM
            for q in range(0, EMBED_DIM, LANES):
                vec = tailv[v, pl.ds(q, LANES)]
                plsc.store_scatter(
                    rout.at[0],
                    [jnp.full((LANES,), r, jnp.int32), _IOTA + (cb + q)],
                    vec)
            return carry

        lax.fori_loop(0, TAILW, v_loop, 0)
        pltpu.sync_copy(rout.at[0, pl.ds(0, TAILW // PACK)],
                        table_rm.at[pl.ds(TAIL0 // PACK, TAILW // PACK)])


def _gather_body(idx_hbm, table_rm, out_hbm, idx_all, idx4a, idx4b, g4, t, gsem):
    wid = lax.axis_index("s") * NUM_CORES + lax.axis_index("c")
    _IOTA = lax.iota(jnp.int32, LANES)
    PERM = [(_IOTA + d) % LANES for d in range(LANES)]
    b0 = wid * (2 * W2)
    pltpu.sync_copy(idx_hbm.at[:, pl.ds(b0, 2 * W2)], idx_all)

    def prep(u, b):
        h = lax.shift_right_logical(u, 1)
        boff = (u & 1) * W2
        idx4 = idx4a if b == 0 else idx4b
        for j in range(W2 // LANES):
            vec = idx_all[h, pl.ds(boff + j * LANES, LANES)]
            idx4[pl.ds(j * LANES, LANES)] = lax.shift_right_logical(vec, 2)

    def g_desc(b):
        idx4 = idx4a if b == 0 else idx4b
        return pltpu.make_async_copy(
            table_rm.at[idx4], g4.at[b], gsem.at[b])

    def unit(u, b):
        h = lax.shift_right_logical(u, 1)
        boff = (u & 1) * W2

        def p_loop(p, carry):
            rows = _IOTA + p * LANES
            colb = (idx_all[h, pl.ds(boff + p * LANES, LANES)] & 3) * EMBED_DIM
            for e0 in range(0, EMBED_DIM, LANES):
                for d in range(LANES):
                    epat = PERM[d] + e0
                    vec = plsc.load_gather(g4.at[b], [rows, colb + epat])
                    plsc.store_scatter(t, [epat, rows], vec)
            return carry

        lax.fori_loop(0, W2 // LANES, p_loop, 0)
        pltpu.sync_copy(t, out_hbm.at[h, :, pl.ds(b0 + boff, W2)])

    prep(0, 0)
    g_desc(0).start()
    prep(1, 1)
    g_desc(1).start()

    def upair(uu, carry):
        for b in range(2):
            u = 2 * uu + b
            g_desc(b).wait()
            unit(u, b)

            @pl.when(uu < NUNIT // 2 - 1)
            def _next():
                prep(u + 2, b)
                g_desc(b).start()

        return carry

    lax.fori_loop(0, NUNIT // 2, upair, 0)


@jax.jit
def _embed_lookup(idx_t, tab_t, tail_pad):
    mesh = plsc.VectorSubcoreMesh(core_axis_name="c", subcore_axis_name="s")
    relayout = functools.partial(
        pl.kernel,
        mesh=mesh,
        out_type=jax.ShapeDtypeStruct((RM_ROWS, 128), jnp.float32),
        scratch_types=[
            pltpu.VMEM((2, EMBED_DIM, W1), jnp.float32),
            pltpu.VMEM((2, W1 // PACK, 128), jnp.float32),
            pltpu.VMEM((TAILW, 128), jnp.float32),
            pltpu.VMEM((LANES, LANES), jnp.int32),
            pltpu.SemaphoreType.DMA((2,)),
            pltpu.SemaphoreType.DMA((2,)),
        ],
        compiler_params=pltpu.CompilerParams(needs_layout_passes=False),
    )(_relayout_body)
    table_rm = relayout(tab_t, tail_pad)

    gather = functools.partial(
        pl.kernel,
        mesh=mesh,
        out_type=jax.ShapeDtypeStruct((HIST, EMBED_DIM, BATCH), jnp.float32),
        scratch_types=[
            pltpu.VMEM((HIST, 2 * W2), jnp.int32),
            pltpu.VMEM((W2,), jnp.int32),
            pltpu.VMEM((W2,), jnp.int32),
            pltpu.VMEM((2, W2, 128), jnp.float32),
            pltpu.VMEM((EMBED_DIM, W2), jnp.float32),
            pltpu.SemaphoreType.DMA((2,)),
        ],
        compiler_params=pltpu.CompilerParams(needs_layout_passes=False),
    )(_gather_body)
    return gather(idx_t, table_rm)


def kernel(indices, table):
    tail_pad = jnp.pad(table[TAIL0:], ((0, 0), (0, 128 - EMBED_DIM)))
    out_t = _embed_lookup(indices.T, table.T, tail_pad)
    return out_t.transpose(2, 0, 1)
